# Initial kernel scaffold; baseline (speedup 1.0000x reference)
import jax
import jax.numpy as jnp
from jax.experimental import pallas as pl


def kernel(hidden_states, router, gate_up_proj, down_proj):
    raise NotImplementedError("write your pallas kernel here")


# speedup vs baseline: 1.1588x; 1.1588x over previous
import functools

import jax
import jax.numpy as jnp
from jax.experimental import pallas as pl

NUM_EXPERTS = 8
TOP_K = 2
HIDDEN = 1024
INTER = 2048
TOKENS = 2048

TOK_BLK = 512


def _route_body(h_ref, r_ref, w_ref):
    h = h_ref[...]
    r = r_ref[...]
    logits = jax.lax.dot_general(
        h, r, (((1,), (1,)), ((), ())),
        preferred_element_type=jnp.float32,
    )
    iota8 = jax.lax.broadcasted_iota(jnp.int32, logits.shape, 1)
    m1 = jnp.max(logits, axis=1, keepdims=True)
    i1 = jnp.min(jnp.where(logits == m1, iota8, NUM_EXPERTS), axis=1, keepdims=True)
    masked = jnp.where(iota8 == i1, -jnp.inf, logits)
    m2 = jnp.max(masked, axis=1, keepdims=True)
    i2 = jnp.min(jnp.where(masked == m2, iota8, NUM_EXPERTS), axis=1, keepdims=True)
    w1 = 1.0 / (1.0 + jnp.exp(m2 - m1))
    w2 = 1.0 - w1
    w_ref[...] = jnp.where(iota8 == i1, w1, 0.0) + jnp.where(iota8 == i2, w2, 0.0)


def _dense_body(x_ref, gw_ref, dw_ref, w_ref, o_ref):
    j = pl.program_id(1)
    x = x_ref[...].astype(jnp.bfloat16)
    gu = jax.lax.dot_general(
        x, gw_ref[0], (((1,), (1,)), ((), ())),
        preferred_element_type=jnp.float32,
    )
    gate = gu[:, :INTER]
    up = gu[:, INTER:]
    h = (gate * jax.lax.logistic(gate) * up).astype(jnp.bfloat16)
    y = jax.lax.dot_general(
        h, dw_ref[0], (((1,), (1,)), ((), ())),
        preferred_element_type=jnp.float32,
    )
    iota8 = jax.lax.broadcasted_iota(jnp.int32, (TOK_BLK, NUM_EXPERTS), 1)
    wcol = jnp.sum(w_ref[...] * (iota8 == j).astype(jnp.float32), axis=1,
                   keepdims=True)
    y = y * wcol

    @pl.when(j == 0)
    def _init():
        o_ref[...] = y

    @pl.when(j > 0)
    def _acc():
        o_ref[...] += y


@jax.jit
def kernel(hidden_states, router, gate_up_proj, down_proj):
    wts = pl.pallas_call(
        _route_body,
        out_shape=jax.ShapeDtypeStruct((TOKENS, NUM_EXPERTS), jnp.float32),
    )(hidden_states, router)

    gw = gate_up_proj.astype(jnp.bfloat16)
    dw = down_proj.astype(jnp.bfloat16)

    out = pl.pallas_call(
        _dense_body,
        grid=(TOKENS // TOK_BLK, NUM_EXPERTS),
        in_specs=[
            pl.BlockSpec((TOK_BLK, HIDDEN), lambda i, j: (i, 0)),
            pl.BlockSpec((1, 2 * INTER, HIDDEN), lambda i, j: (j, 0, 0)),
            pl.BlockSpec((1, HIDDEN, INTER), lambda i, j: (j, 0, 0)),
            pl.BlockSpec((TOK_BLK, NUM_EXPERTS), lambda i, j: (i, 0)),
        ],
        out_specs=pl.BlockSpec((TOK_BLK, HIDDEN), lambda i, j: (i, 0)),
        out_shape=jax.ShapeDtypeStruct((TOKENS, HIDDEN), jnp.float32),
    )(hidden_states, gw, dw, wts)
    return out
